# Initial kernel scaffold; baseline (speedup 1.0000x reference)
import jax
import jax.numpy as jnp
from jax.experimental import pallas as pl


def kernel(x_user, x_item, edge_index_user_rates_item, edge_index_item_rated_by_user, Wl_rates, bl_rates, Wr_rates, Wl_rev, bl_rev, Wr_rev):
    raise NotImplementedError("write your pallas kernel here")


# speedup vs baseline: 2.9881x; 2.9881x over previous
import functools

import jax
import jax.numpy as jnp
from jax import lax
from jax.experimental import pallas as pl
from jax.experimental.pallas import tpu as pltpu
from jax.experimental.pallas import tpu_sc as plsc

N = 50000
E = 625000
D = 128
W = 32
NCOL = D // W
NPASS = NCOL + 1

NSC = 2
NTILE = 16
CH = 128
TE = 39168
E_PAD = NTILE * TE
NCHUNK = TE // CH
R = 50176
TRASH = N
ROWS_PER_TILE = R // NTILE
ZR = 448


def _sc_scatter_build():
    mesh = plsc.VectorSubcoreMesh(core_axis_name="c", subcore_axis_name="s")
    out_type = [jax.ShapeDtypeStruct((NSC * R, W), jnp.float32)
                for _ in range(NPASS)]
    scratch = [
        pltpu.VMEM((CH,), jnp.int32),
        pltpu.VMEM((CH,), jnp.int32),
        pltpu.VMEM((CH, W), jnp.float32),
        pltpu.VMEM((CH, W), jnp.float32),
        pltpu.VMEM((ZR, W), jnp.float32),
        pltpu.VMEM_SHARED((R, W), jnp.float32),
        pltpu.SemaphoreType.DMA,
    ]

    @functools.partial(
        pl.kernel, out_type=out_type, mesh=mesh, scratch_types=scratch,
        compiler_params=pltpu.CompilerParams(use_tc_tiling_on_sc=False))
    def sc_scatter(xs0, xs1, xs2, xs3, srccat, dstcat,
                   o0, o1, o2, o3, ocnt,
                   sidx, didx, msgs, ones, zbuf, acc, sem):
        cid = lax.axis_index("c")
        sid = lax.axis_index("s")
        ebase = cid * E_PAD + sid * TE

        def fill(buf, nrows, val):
            def body(i, _):
                buf[i, pl.ds(0, 16)] = jnp.full((16,), val, jnp.float32)
                buf[i, pl.ds(16, 16)] = jnp.full((16,), val, jnp.float32)
                return 0
            lax.fori_loop(0, nrows, body, 0)

        fill(zbuf, ZR, 0.0)
        fill(ones, CH, 1.0)

        tables = [xs0, xs1, xs2, xs3, None]
        outs = [o0, o1, o2, o3, ocnt]
        for p in range(NPASS):
            for z in range(ROWS_PER_TILE // ZR):
                pltpu.sync_copy(
                    zbuf, acc.at[pl.ds(sid * ROWS_PER_TILE + z * ZR, ZR)])
            plsc.subcore_barrier()

            def body(g, _):
                off = ebase + g * CH
                pltpu.sync_copy(dstcat.at[pl.ds(off, CH)], didx)
                if p < NCOL:
                    pltpu.sync_copy(srccat.at[pl.ds(off, CH)], sidx)
                    pltpu.async_copy(tables[p].at[sidx], msgs, sem).wait()
                    pltpu.sync_copy(msgs, acc.at[didx], add=True)
                else:
                    pltpu.sync_copy(ones, acc.at[didx], add=True)
                return 0
            lax.fori_loop(0, NCHUNK, body, 0)
            plsc.subcore_barrier()

            pltpu.sync_copy(
                acc.at[pl.ds(sid * ROWS_PER_TILE, ROWS_PER_TILE)],
                outs[p].at[pl.ds(cid * R + sid * ROWS_PER_TILE,
                                 ROWS_PER_TILE)])
            plsc.subcore_barrier()

    return sc_scatter


_sc_scatter = _sc_scatter_build()

TC_BLK = 1000


def _tc_body(s0, s1, s2, s3, cnt, xd, wl, wr, b, out):
    aggr = jnp.concatenate([s0[...], s1[...], s2[...], s3[...]], axis=1)
    c = jnp.maximum(cnt[:, 0:1], 1.0)
    aggr = aggr / c
    out[...] = (jnp.dot(aggr, wl[...], preferred_element_type=jnp.float32)
                + jnp.dot(xd[...], wr[...],
                          preferred_element_type=jnp.float32)
                + b[...])


def _tc_final(s0, s1, s2, s3, cnt, x_dst, Wl, Wr, bl):
    nblk = N // TC_BLK
    col = pl.BlockSpec((TC_BLK, W), lambda i: (i, 0))
    full = pl.BlockSpec((TC_BLK, D), lambda i: (i, 0))
    wspec = pl.BlockSpec((D, D), lambda i: (0, 0))
    bspec = pl.BlockSpec((1, D), lambda i: (0, 0))
    return pl.pallas_call(
        _tc_body,
        grid=(nblk,),
        in_specs=[col, col, col, col, col, full, wspec, wspec, bspec],
        out_specs=full,
        out_shape=jax.ShapeDtypeStruct((N, D), jnp.float32),
    )(s0, s1, s2, s3, cnt, x_dst, Wl, Wr, bl.reshape(1, D))


def kernel(x_user, x_item, edge_index_user_rates_item,
           edge_index_item_rated_by_user,
           Wl_rates, bl_rates, Wr_rates,
           Wl_rev, bl_rev, Wr_rev):
    src0 = edge_index_user_rates_item[0].astype(jnp.int32)
    dst0 = edge_index_user_rates_item[1].astype(jnp.int32)
    src1 = edge_index_item_rated_by_user[0].astype(jnp.int32) + N
    dst1 = edge_index_item_rated_by_user[1].astype(jnp.int32)

    zpad = jnp.zeros((E_PAD - E,), jnp.int32)
    tpad = jnp.full((E_PAD - E,), TRASH, jnp.int32)
    srccat = jnp.concatenate([src0, zpad, src1, zpad])
    dstcat = jnp.concatenate([dst0, tpad, dst1, tpad])
    xs = [jnp.concatenate([x_user[:, c * W:(c + 1) * W],
                           x_item[:, c * W:(c + 1) * W]], axis=0)
          for c in range(NCOL)]

    o0, o1, o2, o3, ocnt = _sc_scatter(xs[0], xs[1], xs[2], xs[3],
                                       srccat, dstcat)

    out_item = _tc_final(o0[:N], o1[:N], o2[:N], o3[:N], ocnt[:N],
                         x_item, Wl_rates, Wr_rates, bl_rates)
    out_user = _tc_final(o0[R:R + N], o1[R:R + N], o2[R:R + N],
                         o3[R:R + N], ocnt[R:R + N],
                         x_user, Wl_rev, Wr_rev, bl_rev)
    return (out_user, out_item)


# speedup vs baseline: 4.0507x; 1.3556x over previous
import functools

import jax
import jax.numpy as jnp
from jax import lax
from jax.experimental import pallas as pl
from jax.experimental.pallas import tpu as pltpu
from jax.experimental.pallas import tpu_sc as plsc

N = 50000
E = 625000
D = 128
W = 32
NCOL = D // W
NPASS = NCOL + 1

NSC = 2
NTILE = 16
CH = 128
IBLK = 12
NBLK = 26
NCHUNK = IBLK * NBLK
TE = NCHUNK * CH
E_PAD = NTILE * TE
ROWS2D = NSC * E_PAD // CH
R = 50176
TRASH = N
ROWS_PER_TILE = R // NTILE

K = 2
SUBS = IBLK // K


def _sc_scatter_build():
    mesh = plsc.VectorSubcoreMesh(core_axis_name="c", subcore_axis_name="s")
    out_type = [jax.ShapeDtypeStruct((NSC * R, W), jnp.float32)
                for _ in range(NPASS)]
    scratch = [
        pltpu.VMEM((IBLK, CH), jnp.int32),
        pltpu.VMEM((IBLK, CH), jnp.int32),
        pltpu.VMEM((IBLK, CH), jnp.int32),
        pltpu.VMEM((IBLK, CH), jnp.int32),
        pltpu.VMEM((K * CH, W), jnp.float32),
        pltpu.VMEM((K * CH, W), jnp.float32),
        pltpu.VMEM((CH, W), jnp.float32),
        pltpu.VMEM_SHARED((R, W), jnp.float32),
        pltpu.SemaphoreType.DMA,
        pltpu.SemaphoreType.DMA,
        pltpu.SemaphoreType.DMA,
    ]

    @functools.partial(
        pl.kernel, out_type=out_type, mesh=mesh, scratch_types=scratch,
        compiler_params=pltpu.CompilerParams(use_tc_tiling_on_sc=False))
    def sc_scatter(xs0, xs1, xs2, xs3, srccat, dstcat, zeros,
                   o0, o1, o2, o3, ocnt,
                   s0, s1, d0, d1, m0, m1, ones, acc,
                   isem, gsem0, gsem1):
        cid = lax.axis_index("c")
        sid = lax.axis_index("s")
        row0 = cid * (E_PAD // CH) + sid * NCHUNK

        def body(i, _):
            ones[i, pl.ds(0, 16)] = jnp.full((16,), 1.0, jnp.float32)
            ones[i, pl.ds(16, 16)] = jnp.full((16,), 1.0, jnp.float32)
            return 0
        lax.fori_loop(0, CH, body, 0)

        sbufs = [s0, s1]
        dbufs = [d0, d1]
        msgs = [m0, m1]
        gsems = [gsem0, gsem1]
        tables = [xs0, xs1, xs2, xs3, None]
        outs = [o0, o1, o2, o3, ocnt]
        span = pl.ds(sid * ROWS_PER_TILE, ROWS_PER_TILE)

        def src_rows(i):
            return srccat.at[pl.ds(row0 + i * IBLK, IBLK)]

        def dst_rows(i):
            return dstcat.at[pl.ds(row0 + i * IBLK, IBLK)]

        def run_pass(table, is_count):
            def fire(sb, s, j):
                for k in range(K):
                    pltpu.async_copy(table.at[sb.at[s * K + k]],
                                     msgs[j].at[pl.ds(k * CH, CH)],
                                     gsems[j])

            def drain(sb, j):
                for k in range(K):
                    pltpu.make_async_copy(
                        table.at[sb.at[0]],
                        msgs[j].at[pl.ds(k * CH, CH)], gsems[j]).wait()

            def block(i, b):
                sb, db = sbufs[b], dbufs[b]
                if is_count:
                    descs = [pltpu.async_copy(
                        ones, acc.at[db.at[r]], gsem0, add=True)
                        for r in range(IBLK)]
                    for dsc in descs:
                        dsc.wait()
                else:
                    fire(sb, 0, 0)
                    fire(sb, 1, 1)
                    for s in range(SUBS):
                        j = s % 2
                        drain(sb, j)
                        for k in range(K):
                            pltpu.sync_copy(
                                msgs[j].at[pl.ds(k * CH, CH)],
                                acc.at[db.at[s * K + k]], add=True)
                        if s + 2 < SUBS:
                            fire(sb, s + 2, j)

                @pl.when(i < NBLK - 1)
                def _():
                    pltpu.make_async_copy(
                        src_rows(i + 1), sbufs[1 - b], isem).wait()
                    pltpu.make_async_copy(
                        dst_rows(i + 1), dbufs[1 - b], isem).wait()

                @pl.when(i + 2 < NBLK)
                def _():
                    pltpu.async_copy(src_rows(i + 2), sb, isem)
                    pltpu.async_copy(dst_rows(i + 2), db, isem)

            pltpu.sync_copy(src_rows(0), s0)
            pltpu.sync_copy(dst_rows(0), d0)
            pltpu.async_copy(src_rows(1), s1, isem)
            pltpu.async_copy(dst_rows(1), d1, isem)

            def loop(it, _):
                block(2 * it, 0)
                block(2 * it + 1, 1)
                return 0
            lax.fori_loop(0, NBLK // 2, loop, 0)

        for p in range(NPASS):
            pltpu.sync_copy(zeros.at[span], acc.at[span])
            plsc.subcore_barrier()
            run_pass(tables[p], p == NCOL)
            plsc.subcore_barrier()
            pltpu.sync_copy(acc.at[span],
                            outs[p].at[pl.ds(cid * R + sid * ROWS_PER_TILE,
                                             ROWS_PER_TILE)])
            plsc.subcore_barrier()

    return sc_scatter


_sc_scatter = _sc_scatter_build()

TC_BLK = 1000


def _tc_body(s0, s1, s2, s3, cnt, xd, wl, wr, b, out):
    aggr = jnp.concatenate([s0[...], s1[...], s2[...], s3[...]], axis=1)
    c = jnp.maximum(cnt[:, 0:1], 1.0)
    aggr = aggr / c
    out[...] = (jnp.dot(aggr, wl[...], preferred_element_type=jnp.float32)
                + jnp.dot(xd[...], wr[...],
                          preferred_element_type=jnp.float32)
                + b[...])


def _tc_final(s0, s1, s2, s3, cnt, x_dst, Wl, Wr, bl):
    nblk = N // TC_BLK
    col = pl.BlockSpec((TC_BLK, W), lambda i: (i, 0))
    full = pl.BlockSpec((TC_BLK, D), lambda i: (i, 0))
    wspec = pl.BlockSpec((D, D), lambda i: (0, 0))
    bspec = pl.BlockSpec((1, D), lambda i: (0, 0))
    return pl.pallas_call(
        _tc_body,
        grid=(nblk,),
        in_specs=[col, col, col, col, col, full, wspec, wspec, bspec],
        out_specs=full,
        out_shape=jax.ShapeDtypeStruct((N, D), jnp.float32),
    )(s0, s1, s2, s3, cnt, x_dst, Wl, Wr, bl.reshape(1, D))


def kernel(x_user, x_item, edge_index_user_rates_item,
           edge_index_item_rated_by_user,
           Wl_rates, bl_rates, Wr_rates,
           Wl_rev, bl_rev, Wr_rev):
    src0 = edge_index_user_rates_item[0].astype(jnp.int32)
    dst0 = edge_index_user_rates_item[1].astype(jnp.int32)
    src1 = edge_index_item_rated_by_user[0].astype(jnp.int32) + N
    dst1 = edge_index_item_rated_by_user[1].astype(jnp.int32)

    zpad = jnp.zeros((E_PAD - E,), jnp.int32)
    tpad = jnp.full((E_PAD - E,), TRASH, jnp.int32)
    srccat = jnp.concatenate([src0, zpad, src1, zpad]).reshape(ROWS2D, CH)
    dstcat = jnp.concatenate([dst0, tpad, dst1, tpad]).reshape(ROWS2D, CH)
    xs = [jnp.concatenate([x_user[:, c * W:(c + 1) * W],
                           x_item[:, c * W:(c + 1) * W]], axis=0)
          for c in range(NCOL)]
    zeros = jnp.zeros((R, W), jnp.float32)

    o0, o1, o2, o3, ocnt = _sc_scatter(xs[0], xs[1], xs[2], xs[3],
                                       srccat, dstcat, zeros)

    out_item = _tc_final(o0[:N], o1[:N], o2[:N], o3[:N], ocnt[:N],
                         x_item, Wl_rates, Wr_rates, bl_rates)
    out_user = _tc_final(o0[R:R + N], o1[R:R + N], o2[R:R + N],
                         o3[R:R + N], ocnt[R:R + N],
                         x_user, Wl_rev, Wr_rev, bl_rev)
    return (out_user, out_item)


# speedup vs baseline: 4.2661x; 1.0532x over previous
import functools

import jax
import jax.numpy as jnp
from jax import lax
from jax.experimental import pallas as pl
from jax.experimental.pallas import tpu as pltpu
from jax.experimental.pallas import tpu_sc as plsc

N = 50000
E = 625000
D = 128
W = 32
NCOL = D // W
NPASS = NCOL + 1

NSC = 2
NTILE = 16
CH = 128
IBLK = 12
NBLK = 26
NCHUNK = IBLK * NBLK
TE = NCHUNK * CH
E_PAD = NTILE * TE
ROWS2D = NSC * E_PAD // CH
R = 50048
TRASH = N
ROWS_PER_TILE = R // NTILE

RING = 6


def _sc_scatter_build():
    mesh = plsc.VectorSubcoreMesh(core_axis_name="c", subcore_axis_name="s")
    out_type = [jax.ShapeDtypeStruct((NSC * R, W), jnp.float32)
                for _ in range(NPASS)]
    scratch = (
        [pltpu.VMEM((IBLK, CH), jnp.int32) for _ in range(4)]
        + [pltpu.VMEM((CH, W), jnp.float32) for _ in range(RING)]
        + [pltpu.VMEM_SHARED((R, W), jnp.float32)]
        + [pltpu.SemaphoreType.DMA for _ in range(RING + 1)]
    )

    @functools.partial(
        pl.kernel, out_type=out_type, mesh=mesh, scratch_types=scratch,
        compiler_params=pltpu.CompilerParams(use_tc_tiling_on_sc=False))
    def sc_scatter(xs0, xs1, xs2, xs3, srccat, dstcat, zeros,
                   o0, o1, o2, o3, ocnt,
                   s0, s1, d0, d1, m0, m1, m2, m3, m4, m5, acc,
                   isem, g0, g1, g2, g3, g4, g5):
        cid = lax.axis_index("c")
        sid = lax.axis_index("s")
        row0 = cid * (E_PAD // CH) + sid * NCHUNK

        sbufs = [s0, s1]
        dbufs = [d0, d1]
        msgs = [m0, m1, m2, m3, m4, m5]
        gsems = [g0, g1, g2, g3, g4, g5]
        tables = [xs0, xs1, xs2, xs3, None]
        outs = [o0, o1, o2, o3, ocnt]
        span = pl.ds(sid * ROWS_PER_TILE, ROWS_PER_TILE)

        def src_rows(i):
            return srccat.at[pl.ds(row0 + i * IBLK, IBLK)]

        def dst_rows(i):
            return dstcat.at[pl.ds(row0 + i * IBLK, IBLK)]

        def run_pass(table, is_count):
            def fire(idxbuf, u, r):
                pltpu.async_copy(table.at[idxbuf.at[u]], msgs[r], gsems[r])

            def wait_gather(idxbuf, r):
                pltpu.make_async_copy(
                    table.at[idxbuf.at[0]], msgs[r], gsems[r]).wait()

            def wait_idx(i):
                @pl.when(i < NBLK - 1)
                def _():
                    pltpu.make_async_copy(src_rows(i + 1), s0, isem).wait()
                    pltpu.make_async_copy(dst_rows(i + 1), d0, isem).wait()

            def block(i, b):
                sb, db = sbufs[b], dbufs[b]
                nb = 1 - b
                if is_count:
                    descs = [pltpu.async_copy(
                        msgs[0], acc.at[db.at[u]], gsems[1 + (u % (RING - 1))],
                        add=True) for u in range(IBLK)]
                    for dsc in descs:
                        dsc.wait()
                    wait_idx(i)
                else:
                    for u in range(IBLK):
                        r = u % RING
                        wait_gather(sb, r)
                        pltpu.sync_copy(msgs[r], acc.at[db.at[u]], add=True)
                        if u + RING < IBLK:
                            fire(sb, u + RING, r)
                        else:
                            if u + RING == IBLK:
                                wait_idx(i)

                            @pl.when(i < NBLK - 1)
                            def _():
                                fire(sbufs[nb], u + RING - IBLK, r)

                @pl.when(i + 2 < NBLK)
                def _():
                    pltpu.async_copy(src_rows(i + 2), sb, isem)
                    pltpu.async_copy(dst_rows(i + 2), db, isem)

            pltpu.sync_copy(src_rows(0), s0)
            pltpu.sync_copy(dst_rows(0), d0)
            pltpu.async_copy(src_rows(1), s1, isem)
            pltpu.async_copy(dst_rows(1), d1, isem)
            if is_count:
                def ob(i, _):
                    msgs[0][i, pl.ds(0, 16)] = jnp.full((16,), 1.0,
                                                        jnp.float32)
                    msgs[0][i, pl.ds(16, 16)] = jnp.full((16,), 1.0,
                                                         jnp.float32)
                    return 0
                lax.fori_loop(0, CH, ob, 0)
            else:
                for r in range(RING):
                    fire(s0, r, r)

            def loop(it, _):
                block(2 * it, 0)
                block(2 * it + 1, 1)
                return 0
            lax.fori_loop(0, NBLK // 2, loop, 0)

        for p in range(NPASS):
            pltpu.sync_copy(zeros.at[span], acc.at[span])
            plsc.subcore_barrier()
            run_pass(tables[p], p == NCOL)
            plsc.subcore_barrier()
            pltpu.sync_copy(acc.at[span],
                            outs[p].at[pl.ds(cid * R + sid * ROWS_PER_TILE,
                                             ROWS_PER_TILE)])
            plsc.subcore_barrier()

    return sc_scatter


_sc_scatter = _sc_scatter_build()

TC_BLK = 1000


def _tc_body(s0, s1, s2, s3, cnt, xd, wl, wr, b, out):
    aggr = jnp.concatenate([s0[...], s1[...], s2[...], s3[...]], axis=1)
    c = jnp.maximum(cnt[:, 0:1], 1.0)
    aggr = aggr / c
    out[...] = (jnp.dot(aggr, wl[...], preferred_element_type=jnp.float32)
                + jnp.dot(xd[...], wr[...],
                          preferred_element_type=jnp.float32)
                + b[...])


def _tc_final(s0, s1, s2, s3, cnt, x_dst, Wl, Wr, bl):
    nblk = N // TC_BLK
    col = pl.BlockSpec((TC_BLK, W), lambda i: (i, 0))
    full = pl.BlockSpec((TC_BLK, D), lambda i: (i, 0))
    wspec = pl.BlockSpec((D, D), lambda i: (0, 0))
    bspec = pl.BlockSpec((1, D), lambda i: (0, 0))
    return pl.pallas_call(
        _tc_body,
        grid=(nblk,),
        in_specs=[col, col, col, col, col, full, wspec, wspec, bspec],
        out_specs=full,
        out_shape=jax.ShapeDtypeStruct((N, D), jnp.float32),
    )(s0, s1, s2, s3, cnt, x_dst, Wl, Wr, bl.reshape(1, D))


def kernel(x_user, x_item, edge_index_user_rates_item,
           edge_index_item_rated_by_user,
           Wl_rates, bl_rates, Wr_rates,
           Wl_rev, bl_rev, Wr_rev):
    src0 = edge_index_user_rates_item[0].astype(jnp.int32)
    dst0 = edge_index_user_rates_item[1].astype(jnp.int32)
    src1 = edge_index_item_rated_by_user[0].astype(jnp.int32) + N
    dst1 = edge_index_item_rated_by_user[1].astype(jnp.int32)

    zpad = jnp.zeros((E_PAD - E,), jnp.int32)
    tpad = jnp.full((E_PAD - E,), TRASH, jnp.int32)
    srccat = jnp.concatenate([src0, zpad, src1, zpad]).reshape(ROWS2D, CH)
    dstcat = jnp.concatenate([dst0, tpad, dst1, tpad]).reshape(ROWS2D, CH)
    xs = [jnp.concatenate([x_user[:, c * W:(c + 1) * W],
                           x_item[:, c * W:(c + 1) * W]], axis=0)
          for c in range(NCOL)]
    zeros = jnp.zeros((R, W), jnp.float32)

    o0, o1, o2, o3, ocnt = _sc_scatter(xs[0], xs[1], xs[2], xs[3],
                                       srccat, dstcat, zeros)

    out_item = _tc_final(o0[:N], o1[:N], o2[:N], o3[:N], ocnt[:N],
                         x_item, Wl_rates, Wr_rates, bl_rates)
    out_user = _tc_final(o0[R:R + N], o1[R:R + N], o2[R:R + N],
                         o3[R:R + N], ocnt[R:R + N],
                         x_user, Wl_rev, Wr_rev, bl_rev)
    return (out_user, out_item)
